# baseline (speedup 1.0000x reference)
import functools

import jax
import jax.numpy as jnp
from jax import lax
from jax.experimental import pallas as pl
from jax.experimental.pallas import tpu as pltpu
from jax.experimental.pallas import tpu_sc as plsc
from jax._src.pallas import mpmd
from jax._src.pallas import core as pallas_core
from jax._src.pallas.mosaic import core as tpu_core

VOCAB = 151936
D_MODEL = 2048
BATCH = 4
SEQ = 2048

NUM_CORES = 2
NUM_SUBCORES = 16
NUM_WORKERS = NUM_CORES * NUM_SUBCORES
TOKENS = BATCH * SEQ
TOK_PER_WORKER = TOKENS // NUM_WORKERS

CHUNK = 8
NCHUNK = TOK_PER_WORKER // CHUNK
NBUF = 4
LAG = 2
NSLOT = 3

_VMESH = plsc.VectorSubcoreMesh(core_axis_name="c", subcore_axis_name="s")
_SMESH = plsc.ScalarSubcoreMesh(axis_name="c", num_cores=NUM_CORES)
_TEC_VMEM = pallas_core.CoreMemorySpace(tpu_core.MemorySpace.VMEM, _VMESH)
_TEC_SEM = pallas_core.CoreMemorySpace(tpu_core.MemorySpace.SEMAPHORE, _VMESH)
_SCS_SEM = pallas_core.CoreMemorySpace(tpu_core.MemorySpace.SEMAPHORE, _SMESH)


def _tec_fn(idx_hbm, table_hbm, out_hbm, idx_v, rows, gsem, osem, shared, dsem, full):
    del dsem
    sid = lax.axis_index("s")
    cid = lax.axis_index("c")
    wid = sid * NUM_CORES + cid
    base = wid * TOK_PER_WORKER

    pltpu.sync_copy(idx_hbm.at[pl.ds(base, TOK_PER_WORKER)], idx_v)

    gh = [None] * NBUF
    oh = [None] * NBUF

    for b in range(NBUF):
        gh[b] = pltpu.async_copy(
            table_hbm.at[idx_v.at[pl.ds(b * CHUNK, CHUNK)]],
            rows.at[b], gsem.at[b],
        )

    for c in range(NCHUNK):
        b = c % NBUF
        gh[b].wait()
        oh[b] = pltpu.async_copy(
            rows.at[b], shared.at[sid, c % NSLOT], osem.at[b]
        )
        j = c - LAG
        n = j + NBUF
        if j >= 0 and n < NCHUNK:
            bb = j % NBUF
            oh[bb].wait()
            pltpu.semaphore_signal(full.at[sid], 1)
            gh[bb] = pltpu.async_copy(
                table_hbm.at[idx_v.at[pl.ds(n * CHUNK, CHUNK)]],
                rows.at[bb], gsem.at[bb],
            )

    for c in range(NCHUNK - NBUF, NCHUNK):
        oh[c % NBUF].wait()
        pltpu.semaphore_signal(full.at[sid], 1)


def _scs_fn(idx_hbm, table_hbm, out_hbm, idx_v, rows, gsem, osem, shared, dsem, full):
    del idx_hbm, table_hbm, idx_v, rows, gsem, osem
    cid = lax.axis_index("c")
    dh = [[None] * NUM_SUBCORES for _ in range(NCHUNK)]
    for c in range(NCHUNK):
        slot = c % NSLOT
        for t in range(NUM_SUBCORES):
            pl.semaphore_wait(full.at[t], 1)
            row0 = (t * NUM_CORES + cid) * TOK_PER_WORKER + c * CHUNK
            dh[c][t] = pltpu.async_copy(
                shared.at[t, slot], out_hbm.at[pl.ds(row0, CHUNK)], dsem.at[t]
            )
        jc = c - (NSLOT - 1)
        if jc >= 0:
            for t in range(NUM_SUBCORES):
                dh[jc][t].wait()
    for jc in range(NCHUNK - NSLOT + 1, NCHUNK):
        for t in range(NUM_SUBCORES):
            dh[jc][t].wait()


_embed_sc = mpmd.mpmd_map(
    [(_SMESH, _scs_fn), (_VMESH, _tec_fn)],
    out_types=jax.ShapeDtypeStruct((TOKENS, D_MODEL), jnp.float32),
    scratch_types=(
        _TEC_VMEM((TOK_PER_WORKER,), jnp.int32),
        _TEC_VMEM((NBUF, CHUNK, D_MODEL), jnp.float32),
        _TEC_SEM((NBUF,), pltpu.SemaphoreType.DMA.dtype),
        _TEC_SEM((NBUF,), pltpu.SemaphoreType.DMA.dtype),
        pltpu.VMEM_SHARED((NUM_SUBCORES, NSLOT, CHUNK, D_MODEL), jnp.float32),
        _SCS_SEM((NUM_SUBCORES,), pltpu.SemaphoreType.DMA.dtype),
        _SCS_SEM((NUM_SUBCORES,), pltpu.SemaphoreType.REGULAR.dtype),
    ),
)


def kernel(input_ids, table):
    flat_ids = input_ids.reshape(TOKENS)
    out = _embed_sc(flat_ids, table)
    return out.reshape(BATCH, SEQ, D_MODEL)


# speedup vs baseline: 1.0951x; 1.0951x over previous
import functools

import jax
import jax.numpy as jnp
from jax import lax
from jax.experimental import pallas as pl
from jax.experimental.pallas import tpu as pltpu
from jax.experimental.pallas import tpu_sc as plsc

VOCAB = 151936
D_MODEL = 2048
BATCH = 4
SEQ = 2048

NUM_CORES = 2
NUM_SUBCORES = 16
NUM_WORKERS = NUM_CORES * NUM_SUBCORES
TOKENS = BATCH * SEQ
TOK_PER_WORKER = TOKENS // NUM_WORKERS

CHUNK = 16
NCHUNK = TOK_PER_WORKER // CHUNK
NBUF = 3

_MESH = plsc.VectorSubcoreMesh(core_axis_name="c", subcore_axis_name="s")


@functools.partial(
    pl.kernel,
    out_type=jax.ShapeDtypeStruct((TOKENS, D_MODEL), jnp.float32),
    mesh=_MESH,
    scratch_types=(
        [pltpu.VMEM((TOK_PER_WORKER,), jnp.int32)]
        + [pltpu.VMEM((CHUNK, D_MODEL), jnp.float32) for _ in range(NBUF)]
        + [pltpu.SemaphoreType.DMA for _ in range(NBUF)]
        + [pltpu.SemaphoreType.DMA for _ in range(NBUF)]
    ),
)
def _embed_sc(idx_hbm, table_hbm, out_hbm, idx_v, *bufs_and_sems):
    rows = list(bufs_and_sems[:NBUF])
    gsem = list(bufs_and_sems[NBUF:2 * NBUF])
    osem = list(bufs_and_sems[2 * NBUF:3 * NBUF])

    wid = lax.axis_index("s") * NUM_CORES + lax.axis_index("c")
    base = wid * TOK_PER_WORKER

    pltpu.sync_copy(idx_hbm.at[pl.ds(base, TOK_PER_WORKER)], idx_v)

    gh = [None] * NBUF
    oh = [None] * NBUF

    for b in range(NBUF):
        gh[b] = pltpu.async_copy(
            table_hbm.at[idx_v.at[pl.ds(b * CHUNK, CHUNK)]], rows[b], gsem[b]
        )

    for c in range(NCHUNK):
        b = c % NBUF
        gh[b].wait()
        oh[b] = pltpu.async_copy(
            rows[b], out_hbm.at[pl.ds(base + c * CHUNK, CHUNK)], osem[b]
        )
        nxt = c + NBUF
        if nxt < NCHUNK:
            oh[b].wait()
            gh[b] = pltpu.async_copy(
                table_hbm.at[idx_v.at[pl.ds(nxt * CHUNK, CHUNK)]],
                rows[b],
                gsem[b],
            )

    for c in range(NCHUNK - NBUF, NCHUNK):
        if c >= 0:
            oh[c % NBUF].wait()


def kernel(input_ids, table):
    flat_ids = input_ids.reshape(TOKENS)
    out = _embed_sc(flat_ids, table)
    return out.reshape(BATCH, SEQ, D_MODEL)
